# baseline (speedup 1.0000x reference)
import jax
import jax.numpy as jnp
from jax.experimental import pallas as pl

_PER_PROG = 8
_DOT_MSG = (((1,), (1,)), ((0,), (0,)))
_DOT_XW = (((2,), (0,)), ((), ()))


def _split_bf16(v):
    hi = v.astype(jnp.bfloat16)
    lo = (v - hi.astype(jnp.float32)).astype(jnp.bfloat16)
    return hi, lo


def _gcn_body(x_ref, adj_ref, w1_ref, b1_ref, w2_ref, b2_ref, out_ref):
    A = adj_ref[...]
    X = x_ref[...]
    deg = jnp.sum(A.astype(jnp.float32), axis=1) + 1.0
    dinv = jnp.where(deg > 0, jax.lax.rsqrt(deg), 0.0)[..., None]

    def layer(h, W, b):
        t = dinv * jax.lax.dot_general(h, W, _DOT_XW,
                                       preferred_element_type=jnp.float32)
        t_hi, t_lo = _split_bf16(t)
        msg = (jax.lax.dot_general(A, t_hi, _DOT_MSG,
                                   preferred_element_type=jnp.float32)
               + jax.lax.dot_general(A, t_lo, _DOT_MSG,
                                     preferred_element_type=jnp.float32))
        return dinv * (msg + t) + b

    h = jax.nn.relu(layer(X, w1_ref[...], b1_ref[...]))
    h = layer(h, w2_ref[...], b2_ref[...])
    out_ref[0] = jnp.mean(h, axis=1)


def kernel(x, adj, W1, b1, W2, b2):
    B, N, F = x.shape
    O = W2.shape[1]
    g = _PER_PROG
    return pl.pallas_call(
        _gcn_body,
        grid=(B // g,),
        in_specs=[
            pl.BlockSpec((g, N, F), lambda b: (b, 0, 0)),
            pl.BlockSpec((g, N, N), lambda b: (b, 0, 0)),
            pl.BlockSpec(W1.shape, lambda b: (0, 0)),
            pl.BlockSpec((1, b1.shape[0]), lambda b: (0, 0)),
            pl.BlockSpec(W2.shape, lambda b: (0, 0)),
            pl.BlockSpec((1, b2.shape[0]), lambda b: (0, 0)),
        ],
        out_specs=pl.BlockSpec((1, g, O), lambda b: (b, 0, 0)),
        out_shape=jax.ShapeDtypeStruct((B // g, g, O), x.dtype),
    )(x, adj.astype(jnp.bfloat16), W1, b1.reshape(1, -1), W2,
      b2.reshape(1, -1)).reshape(B, O)


# speedup vs baseline: 1.5348x; 1.5348x over previous
import jax
import jax.numpy as jnp
from jax.experimental import pallas as pl

_PER_PROG = 8
_DOT_MSG = (((1,), (1,)), ((0,), (0,)))
_DOT_XW = (((2,), (0,)), ((), ()))


def _split_bf16(v):
    hi = v.astype(jnp.bfloat16)
    lo = (v - hi.astype(jnp.float32)).astype(jnp.bfloat16)
    return hi, lo


def _gcn_body(x_ref, adj_ref, w1_ref, b1_ref, w2_ref, b2_ref, out_ref):
    A32 = adj_ref[...]
    A = A32.astype(jnp.bfloat16)
    X = x_ref[...]
    deg = jnp.sum(A32, axis=1) + 1.0
    dinv = jnp.where(deg > 0, jax.lax.rsqrt(deg), 0.0)[..., None]

    def layer(h, W, b):
        t = dinv * jax.lax.dot_general(h, W, _DOT_XW,
                                       preferred_element_type=jnp.float32)
        t_hi, t_lo = _split_bf16(t)
        msg = (jax.lax.dot_general(A, t_hi, _DOT_MSG,
                                   preferred_element_type=jnp.float32)
               + jax.lax.dot_general(A, t_lo, _DOT_MSG,
                                     preferred_element_type=jnp.float32))
        return dinv * (msg + t) + b

    h = jax.nn.relu(layer(X, w1_ref[...], b1_ref[...]))
    h = layer(h, w2_ref[...], b2_ref[...])
    out_ref[0] = jnp.mean(h, axis=1)


def kernel(x, adj, W1, b1, W2, b2):
    B, N, F = x.shape
    O = W2.shape[1]
    g = _PER_PROG
    return pl.pallas_call(
        _gcn_body,
        grid=(B // g,),
        in_specs=[
            pl.BlockSpec((g, N, F), lambda b: (b, 0, 0)),
            pl.BlockSpec((g, N, N), lambda b: (b, 0, 0)),
            pl.BlockSpec(W1.shape, lambda b: (0, 0)),
            pl.BlockSpec((1, b1.shape[0]), lambda b: (0, 0)),
            pl.BlockSpec(W2.shape, lambda b: (0, 0)),
            pl.BlockSpec((1, b2.shape[0]), lambda b: (0, 0)),
        ],
        out_specs=pl.BlockSpec((1, g, O), lambda b: (b, 0, 0)),
        out_shape=jax.ShapeDtypeStruct((B // g, g, O), x.dtype),
    )(x, adj, W1, b1.reshape(1, -1), W2,
      b2.reshape(1, -1)).reshape(B, O)


# speedup vs baseline: 1.8671x; 1.2166x over previous
import jax
import jax.numpy as jnp
from jax.experimental import pallas as pl
from jax.experimental.pallas import tpu as pltpu

_PER_PROG = 8
_DOT_MSG = (((1,), (1,)), ((0,), (0,)))
_DOT_XW = (((2,), (0,)), ((), ()))


def _gcn_body(x_ref, adj_ref, w1_ref, b1_ref, w2_ref, b2_ref, out_ref):
    A = adj_ref[...]
    X = x_ref[...]
    deg = jnp.sum(A, axis=1) + 1.0
    dinv = jnp.where(deg > 0, jax.lax.rsqrt(deg), 0.0)[..., None]

    def layer(h, W, b):
        t = dinv * jax.lax.dot_general(h, W, _DOT_XW,
                                       preferred_element_type=jnp.float32)
        msg = jax.lax.dot_general(A, t, _DOT_MSG,
                                  preferred_element_type=jnp.float32)
        return dinv * (msg + t) + b

    h = jax.nn.relu(layer(X, w1_ref[...], b1_ref[...]))
    h = layer(h, w2_ref[...], b2_ref[...])
    out_ref[0] = jnp.mean(h, axis=1)


def kernel(x, adj, W1, b1, W2, b2):
    B, N, F = x.shape
    O = W2.shape[1]
    g = _PER_PROG
    return pl.pallas_call(
        _gcn_body,
        grid=(B // g,),
        in_specs=[
            pl.BlockSpec((g, N, F), lambda b: (b, 0, 0)),
            pl.BlockSpec((g, N, N), lambda b: (b, 0, 0)),
            pl.BlockSpec(W1.shape, lambda b: (0, 0)),
            pl.BlockSpec((1, b1.shape[0]), lambda b: (0, 0)),
            pl.BlockSpec(W2.shape, lambda b: (0, 0)),
            pl.BlockSpec((1, b2.shape[0]), lambda b: (0, 0)),
        ],
        out_specs=pl.BlockSpec((1, g, O), lambda b: (b, 0, 0)),
        out_shape=jax.ShapeDtypeStruct((B // g, g, O), x.dtype),
        compiler_params=pltpu.CompilerParams(
            dimension_semantics=("parallel",)),
    )(x, adj, W1, b1.reshape(1, -1), W2,
      b2.reshape(1, -1)).reshape(B, O)
